# baseline (speedup 1.0000x reference)
import functools

import jax
import jax.numpy as jnp
from jax import lax
from jax.experimental import pallas as pl
from jax.experimental.pallas import tpu as pltpu
from jax.experimental.pallas import tpu_sc as plsc

_N_STIM = 21
_NPAIR = _N_STIM * _N_STIM
_NPAD = 448
_B = 16384
_NC = 1
_NS = 16
_NW = _NC * _NS
_L = 16
_BPW = _B // _NW
_G = _BPW // _L
_TG = _NPAD // _L


def _sc_rank(idx_hbm, tbl_hbm, out_hbm, idx_v, tbl_v, stab_v, out_v,
             stab_sh, sem):
    wid = lax.axis_index("s") * _NC + lax.axis_index("c")
    idx_cp = pltpu.async_copy(
        idx_hbm.at[pl.ds(wid * (_BPW * 3), _BPW * 3)], idx_v, sem)
    pltpu.sync_copy(tbl_hbm, tbl_v)
    lanes = lax.iota(jnp.int32, _L)

    def mk_stab(g, carry):
        f = jnp.minimum(g * _L + lanes, _NPAIR - 1)
        q = lax.shift_right_logical(f * 3121, 16)
        r = f - q * _N_STIM
        q3 = q * 3
        r3 = r * 3
        d2 = jnp.zeros((_L,), jnp.float32)
        for k in range(3):
            diff = plsc.load_gather(tbl_v, [q3 + k]) - plsc.load_gather(
                tbl_v, [r3 + k])
            d2 = d2 + diff * diff
        d2s = jnp.maximum(d2, 1e-30)
        bits = plsc.bitcast(d2s, jnp.int32)
        y = plsc.bitcast(0x5F3759DF - lax.shift_right_logical(bits, 1),
                         jnp.float32)
        half = -0.5 * d2s
        for _ in range(2):
            y = y * (1.5 + half * y * y)
        d = d2 * y
        stab_v[pl.ds(g * _L, _L)] = jnp.exp(-10.0 * d) + 0.001
        return carry

    mk_stab(wid, 0)
    pltpu.sync_copy(stab_v.at[pl.ds(wid * _L, _L)],
                    stab_sh.at[pl.ds(wid * _L, _L)])

    @pl.when(wid < _TG - _NW)
    def _second_group():
        g2 = wid + _NW
        mk_stab(g2, 0)
        pltpu.sync_copy(stab_v.at[pl.ds(g2 * _L, _L)],
                        stab_sh.at[pl.ds(g2 * _L, _L)])

    plsc.subcore_barrier()
    pltpu.sync_copy(stab_sh, stab_v)
    idx_cp.wait()

    def body(g, carry):
        r3, o2 = carry
        q = plsc.load_gather(idx_v, [r3])
        r1 = plsc.load_gather(idx_v, [r3 + 1])
        r2 = plsc.load_gather(idx_v, [r3 + 2])
        q21 = q * _N_STIM
        s1 = plsc.load_gather(stab_v, [q21 + r1])
        s2 = plsc.load_gather(stab_v, [q21 + r2])
        p0 = s1 / (s1 + s2)
        plsc.store_scatter(out_v, [o2], p0)
        plsc.store_scatter(out_v, [o2 + 1], 1.0 - p0)
        return (r3 + 3 * _L, o2 + 2 * _L)

    n_chunks = 4
    gpc = _G // n_chunks
    wpc = gpc * _L * 2
    carry = (lanes * 3, lanes * 2)
    cps = []
    for c in range(n_chunks):
        carry = lax.fori_loop(c * gpc, (c + 1) * gpc, body, carry, unroll=4)
        cps.append(pltpu.async_copy(
            out_v.at[pl.ds(c * wpc, wpc)],
            out_hbm.at[pl.ds(wid * (_BPW * 2) + c * wpc, wpc)], sem))
    for cp in cps:
        cp.wait()


@functools.cache
def _sc_rank_call():
    mesh = plsc.VectorSubcoreMesh(
        core_axis_name="c", subcore_axis_name="s", num_cores=_NC)
    return pl.kernel(
        _sc_rank,
        out_type=jax.ShapeDtypeStruct((_B * 2,), jnp.float32),
        mesh=mesh,
        compiler_params=pltpu.CompilerParams(needs_layout_passes=False),
        scratch_types=[
            pltpu.VMEM((_BPW * 3,), jnp.int32),
            pltpu.VMEM((_N_STIM * 3,), jnp.float32),
            pltpu.VMEM((_NPAD,), jnp.float32),
            pltpu.VMEM((_BPW * 2,), jnp.float32),
            pltpu.VMEM_SHARED((_NPAD,), jnp.float32),
            pltpu.SemaphoreType.DMA,
        ],
    )


def kernel(given2rank1_stimulus_set, percept_table):
    tbl = percept_table.astype(jnp.float32).reshape(_N_STIM * 3)
    idx = given2rank1_stimulus_set.astype(jnp.int32).reshape(_B * 3)
    return _sc_rank_call()(idx, tbl).reshape(_B, 2)


# speedup vs baseline: 1.0001x; 1.0001x over previous
import functools

import jax
import jax.numpy as jnp
from jax import lax
from jax.experimental import pallas as pl
from jax.experimental.pallas import tpu as pltpu
from jax.experimental.pallas import tpu_sc as plsc

_N_STIM = 21
_NPAIR = _N_STIM * _N_STIM
_NPAD = 448
_B = 16384
_NC = 1
_NS = 16
_NW = _NC * _NS
_L = 16
_BPW = _B // _NW
_G = _BPW // _L
_TG = _NPAD // _L


def _sc_rank(idx_hbm, tbl_hbm, out_hbm, idx_v, tbl_v, stab_v, out_v,
             stab_sh, sem):
    wid = lax.axis_index("s") * _NC + lax.axis_index("c")
    idx_cp = pltpu.async_copy(
        idx_hbm.at[pl.ds(wid * (_BPW * 3), _BPW * 3)], idx_v, sem)
    pltpu.sync_copy(tbl_hbm, tbl_v)
    lanes = lax.iota(jnp.int32, _L)

    def mk_stab(g, carry):
        f = jnp.minimum(g * _L + lanes, _NPAIR - 1)
        q = lax.shift_right_logical(f * 3121, 16)
        r = f - q * _N_STIM
        q3 = q * 3
        r3 = r * 3
        d2 = jnp.zeros((_L,), jnp.float32)
        for k in range(3):
            diff = plsc.load_gather(tbl_v, [q3 + k]) - plsc.load_gather(
                tbl_v, [r3 + k])
            d2 = d2 + diff * diff
        d2s = jnp.maximum(d2, 1e-30)
        bits = plsc.bitcast(d2s, jnp.int32)
        y = plsc.bitcast(0x5F3759DF - lax.shift_right_logical(bits, 1),
                         jnp.float32)
        half = -0.5 * d2s
        for _ in range(2):
            y = y * (1.5 + half * y * y)
        d = d2 * y
        stab_v[pl.ds(g * _L, _L)] = jnp.exp(-10.0 * d) + 0.001
        return carry

    mk_stab(wid, 0)
    pltpu.sync_copy(stab_v.at[pl.ds(wid * _L, _L)],
                    stab_sh.at[pl.ds(wid * _L, _L)])

    @pl.when(wid < _TG - _NW)
    def _second_group():
        g2 = wid + _NW
        mk_stab(g2, 0)
        pltpu.sync_copy(stab_v.at[pl.ds(g2 * _L, _L)],
                        stab_sh.at[pl.ds(g2 * _L, _L)])

    plsc.subcore_barrier()
    pltpu.sync_copy(stab_sh, stab_v)
    idx_cp.wait()

    def body(g, carry):
        rows = g * _L + lanes
        r3 = rows * 3
        o2 = rows * 2
        q = plsc.load_gather(idx_v, [r3])
        r1 = plsc.load_gather(idx_v, [r3 + 1])
        r2 = plsc.load_gather(idx_v, [r3 + 2])
        q21 = q * _N_STIM
        s1 = plsc.load_gather(stab_v, [q21 + r1])
        s2 = plsc.load_gather(stab_v, [q21 + r2])
        p0 = s1 / (s1 + s2)
        plsc.store_scatter(out_v, [o2], p0)
        plsc.store_scatter(out_v, [o2 + 1], 1.0 - p0)
        return carry

    n_chunks = 4
    gpc = _G // n_chunks
    wpc = gpc * _L * 2
    cps = []
    for c in range(n_chunks):
        lax.fori_loop(c * gpc, (c + 1) * gpc, body, 0, unroll=4)
        cps.append(pltpu.async_copy(
            out_v.at[pl.ds(c * wpc, wpc)],
            out_hbm.at[pl.ds(wid * (_BPW * 2) + c * wpc, wpc)], sem))
    for cp in cps:
        cp.wait()


@functools.cache
def _sc_rank_call():
    mesh = plsc.VectorSubcoreMesh(
        core_axis_name="c", subcore_axis_name="s", num_cores=_NC)
    return pl.kernel(
        _sc_rank,
        out_type=jax.ShapeDtypeStruct((_B * 2,), jnp.float32),
        mesh=mesh,
        compiler_params=pltpu.CompilerParams(needs_layout_passes=False),
        scratch_types=[
            pltpu.VMEM((_BPW * 3,), jnp.int32),
            pltpu.VMEM((_N_STIM * 3,), jnp.float32),
            pltpu.VMEM((_NPAD,), jnp.float32),
            pltpu.VMEM((_BPW * 2,), jnp.float32),
            pltpu.VMEM_SHARED((_NPAD,), jnp.float32),
            pltpu.SemaphoreType.DMA,
        ],
    )


def kernel(given2rank1_stimulus_set, percept_table):
    tbl = percept_table.astype(jnp.float32).reshape(_N_STIM * 3)
    idx = given2rank1_stimulus_set.astype(jnp.int32).reshape(_B * 3)
    return _sc_rank_call()(idx, tbl).reshape(_B, 2)


# speedup vs baseline: 1.0040x; 1.0039x over previous
import functools

import jax
import jax.numpy as jnp
from jax import lax
from jax.experimental import pallas as pl
from jax.experimental.pallas import tpu as pltpu
from jax.experimental.pallas import tpu_sc as plsc

_N_STIM = 21
_NPAIR = _N_STIM * _N_STIM
_NPAD = 448
_B = 16384
_NC = 1
_NS = 16
_NW = _NC * _NS
_L = 16
_BPW = _B // _NW
_G = _BPW // _L
_TG = _NPAD // _L


def _sc_rank(idx_hbm, tbl_hbm, out_hbm, idx_v, tbl_v, stab_v, out_v,
             stab_sh, sem):
    wid = lax.axis_index("s") * _NC + lax.axis_index("c")
    idx_cp = pltpu.async_copy(
        idx_hbm.at[pl.ds(wid * (_BPW * 3), _BPW * 3)], idx_v, sem)
    pltpu.sync_copy(tbl_hbm, tbl_v)
    lanes = lax.iota(jnp.int32, _L)

    def mk_stab(g, carry):
        f = jnp.minimum(g * _L + lanes, _NPAIR - 1)
        q = lax.shift_right_logical(f * 3121, 16)
        r = f - q * _N_STIM
        q3 = q * 3
        r3 = r * 3
        d2 = jnp.zeros((_L,), jnp.float32)
        for k in range(3):
            diff = plsc.load_gather(tbl_v, [q3 + k]) - plsc.load_gather(
                tbl_v, [r3 + k])
            d2 = d2 + diff * diff
        d2s = jnp.maximum(d2, 1e-30)
        bits = plsc.bitcast(d2s, jnp.int32)
        y = plsc.bitcast(0x5F3759DF - lax.shift_right_logical(bits, 1),
                         jnp.float32)
        half = -0.5 * d2s
        for _ in range(2):
            y = y * (1.5 + half * y * y)
        d = d2 * y
        stab_v[pl.ds(g * _L, _L)] = jnp.exp(-10.0 * d) + 0.001
        return carry

    mk_stab(wid, 0)
    pltpu.sync_copy(stab_v.at[pl.ds(wid * _L, _L)],
                    stab_sh.at[pl.ds(wid * _L, _L)])

    @pl.when(wid < _TG - _NW)
    def _second_group():
        g2 = wid + _NW
        mk_stab(g2, 0)
        pltpu.sync_copy(stab_v.at[pl.ds(g2 * _L, _L)],
                        stab_sh.at[pl.ds(g2 * _L, _L)])

    plsc.subcore_barrier()
    pltpu.sync_copy(stab_sh, stab_v)
    idx_cp.wait()

    def body(g, carry):
        rows = g * _L + lanes
        r3 = rows * 3
        q = plsc.load_gather(idx_v, [r3])
        r1 = plsc.load_gather(idx_v, [r3 + 1])
        r2 = plsc.load_gather(idx_v, [r3 + 2])
        q21 = q * _N_STIM
        s1 = plsc.load_gather(stab_v, [q21 + r1])
        s2 = plsc.load_gather(stab_v, [q21 + r2])
        p0 = s1 / (s1 + s2)
        o2 = rows * 2
        plsc.store_scatter(out_v, [o2], p0)
        plsc.store_scatter(out_v, [o2 + 1], 1.0 - p0)
        return carry

    half_w = _BPW
    lax.fori_loop(0, _G // 2, body, 0, unroll=4)
    out_cp0 = pltpu.async_copy(
        out_v.at[pl.ds(0, half_w)],
        out_hbm.at[pl.ds(wid * (_BPW * 2), half_w)], sem)
    lax.fori_loop(_G // 2, _G, body, 0, unroll=4)
    out_cp1 = pltpu.async_copy(
        out_v.at[pl.ds(half_w, half_w)],
        out_hbm.at[pl.ds(wid * (_BPW * 2) + half_w, half_w)], sem)
    out_cp0.wait()
    out_cp1.wait()


@functools.cache
def _sc_rank_call():
    mesh = plsc.VectorSubcoreMesh(
        core_axis_name="c", subcore_axis_name="s", num_cores=_NC)
    return pl.kernel(
        _sc_rank,
        out_type=jax.ShapeDtypeStruct((_B * 2,), jnp.float32),
        mesh=mesh,
        compiler_params=pltpu.CompilerParams(needs_layout_passes=False),
        scratch_types=[
            pltpu.VMEM((_BPW * 3,), jnp.int32),
            pltpu.VMEM((_N_STIM * 3,), jnp.float32),
            pltpu.VMEM((_NPAD,), jnp.float32),
            pltpu.VMEM((_BPW * 2,), jnp.float32),
            pltpu.VMEM_SHARED((_NPAD,), jnp.float32),
            pltpu.SemaphoreType.DMA,
        ],
    )


def kernel(given2rank1_stimulus_set, percept_table):
    tbl = percept_table.astype(jnp.float32).reshape(_N_STIM * 3)
    idx = given2rank1_stimulus_set.astype(jnp.int32).reshape(_B * 3)
    return _sc_rank_call()(idx, tbl).reshape(_B, 2)
